# Initial kernel scaffold; baseline (speedup 1.0000x reference)
import functools

import jax
import jax.numpy as jnp
from jax import lax
from jax.experimental import pallas as pl
from jax.experimental.pallas import tpu as pltpu
from jax.experimental.pallas import tpu_sc as plsc

_N = 10000
_T = 8
_E = 320000
_D = 128
_H1 = 64
_HG = 32
_WIN = 4

_NC = 2
_NS = 16
_NW = _NC * _NS
_CH = 128
_CHT = 79
_EP = _NW * _CHT * _CH
_DCH = (_T * _E) // (_NW * _CH)
_DSLAB = 125


def _sc_deg(dstf, wf, z1):
    mesh = plsc.VectorSubcoreMesh(core_axis_name="c", subcore_axis_name="s")

    @functools.partial(
        pl.kernel,
        mesh=mesh,
        out_type=jax.ShapeDtypeStruct((_NC, _T * _N), jnp.float32),
        scratch_types=[
            pltpu.VMEM((_DSLAB, _CH), jnp.int32),
            pltpu.VMEM((_DSLAB, _CH), jnp.float32),
            pltpu.VMEM_SHARED((_T * _N,), jnp.float32),
        ],
    )
    def k(dst_hbm, w_hbm, z_hbm, out_hbm, dbuf, vbuf, acc):
        c = lax.axis_index("c")
        s = lax.axis_index("s")
        wid = s * _NC + c
        spt = (_T * _N) // _NS
        pltpu.sync_copy(z_hbm.at[pl.ds(s * spt, spt)], acc.at[pl.ds(s * spt, spt)])
        plsc.subcore_barrier()
        for slab in range(_DCH // _DSLAB):
            pltpu.sync_copy(dst_hbm.at[wid, pl.ds(slab * _DSLAB, _DSLAB)], dbuf)
            pltpu.sync_copy(w_hbm.at[wid, pl.ds(slab * _DSLAB, _DSLAB)], vbuf)

            def body(j, carry):
                pltpu.sync_copy(vbuf.at[j], acc.at[dbuf.at[j]], add=True)
                return carry

            lax.fori_loop(0, _DSLAB, body, 0)
        plsc.subcore_barrier()
        pltpu.sync_copy(acc.at[pl.ds(s * spt, spt)], out_hbm.at[c, pl.ds(s * spt, spt)])

    return k(dstf, wf, z1)


def _sc_agg(xs, src, dst, w, zrows):
    mesh = plsc.VectorSubcoreMesh(core_axis_name="c", subcore_axis_name="s")

    @functools.partial(
        pl.kernel,
        mesh=mesh,
        out_type=jax.ShapeDtypeStruct((_NC, _N, _H1), jnp.float32),
        scratch_types=[
            pltpu.VMEM((_CHT, _CH), jnp.int32),
            pltpu.VMEM((_CHT, _CH), jnp.int32),
            pltpu.VMEM((_CHT, _CH), jnp.float32),
            pltpu.VMEM((_CH, _H1), jnp.float32),
            pltpu.VMEM_SHARED((_N, _H1), jnp.float32),
            pltpu.SemaphoreType.DMA,
        ],
    )
    def k(xs_hbm, src_hbm, dst_hbm, w_hbm, z_hbm, out_hbm,
          srcb, dstb, wb, rows, acc, sem):
        c = lax.axis_index("c")
        s = lax.axis_index("s")
        wid = s * _NC + c
        rpt = _N // _NS
        pltpu.sync_copy(src_hbm.at[wid], srcb)
        pltpu.sync_copy(dst_hbm.at[wid], dstb)
        pltpu.sync_copy(w_hbm.at[wid], wb)
        pltpu.sync_copy(z_hbm.at[pl.ds(s * rpt, rpt)], acc.at[pl.ds(s * rpt, rpt)])
        plsc.subcore_barrier()

        def body(j, carry):
            pltpu.async_copy(xs_hbm.at[srcb.at[j]], rows, sem).wait()

            def scale(e, c2):
                ws = wb[j, e]
                for i in range(_H1 // 16):
                    sl = pl.ds(i * 16, 16)
                    rows[e, sl] = rows[e, sl] * ws
                return c2

            lax.fori_loop(0, _CH, scale, 0)
            pltpu.sync_copy(rows, acc.at[dstb.at[j]], add=True)
            return carry

        lax.fori_loop(0, _CHT, body, 0)
        plsc.subcore_barrier()
        pltpu.sync_copy(acc.at[pl.ds(s * rpt, rpt)],
                        out_hbm.at[c, pl.ds(s * rpt, rpt)])

    return k(xs, src, dst, w, zrows)


def _tc_pre(x, degp, W1):
    def body(x_ref, degp_ref, w1_ref, xs_ref, dis_ref):
        xt = x_ref[0]
        xw = lax.dot_general(xt, w1_ref[...], (((0,), (0,)), ((), ())),
                             preferred_element_type=jnp.float32)
        deg = degp_ref[0, 0] + degp_ref[1, 0] + 1.0
        dis = lax.rsqrt(deg)
        dis_ref[0] = dis
        xs_ref[0] = xw * dis

    return pl.pallas_call(
        body,
        grid=(_T,),
        in_specs=[
            pl.BlockSpec((1, _D, _N), lambda t: (t, 0, 0)),
            pl.BlockSpec((_NC, 1, _N, 1), lambda t: (0, t, 0, 0)),
            pl.BlockSpec((_D, _H1), lambda t: (0, 0)),
        ],
        out_specs=[
            pl.BlockSpec((1, _N, _H1), lambda t: (t, 0, 0)),
            pl.BlockSpec((1, _N, 1), lambda t: (t, 0, 0)),
        ],
        out_shape=[
            jax.ShapeDtypeStruct((_T, _N, _H1), jnp.float32),
            jax.ShapeDtypeStruct((_T, _N, 1), jnp.float32),
        ],
    )(x, degp, W1)


def _tc_mid(parts, xs1, dis, b1, W2):
    def body(p_ref, xs_ref, dis_ref, b_ref, w2_ref, o_ref):
        psum = p_ref[0] + p_ref[1] + xs_ref[...]
        out1 = jnp.maximum(psum * dis_ref[...] + b_ref[...], 0.0)
        xw2 = lax.dot_general(out1, w2_ref[...], (((1,), (0,)), ((), ())),
                              preferred_element_type=jnp.float32)
        o_ref[...] = xw2 * dis_ref[...]

    return pl.pallas_call(
        body,
        out_shape=jax.ShapeDtypeStruct((_N, _H1), jnp.float32),
    )(parts, xs1, dis, b1, W2)


def _tc_post(parts, xs2, dis, b2):
    def body(p_ref, xs_ref, dis_ref, b_ref, o_ref):
        psum = p_ref[0] + p_ref[1] + xs_ref[...]
        o_ref[...] = jnp.maximum(psum * dis_ref[...] + b_ref[...], 0.0)

    return pl.pallas_call(
        body,
        out_shape=jax.ShapeDtypeStruct((_N, _H1), jnp.float32),
    )(parts, xs2, dis, b2)


def _tc_gru(gcn, wihT, whhT, bih, bhh, wpT, bp):
    BN = 2500

    def body(g_ref, wih_ref, whh_ref, bih_ref, bhh_ref, wp_ref, bp_ref, o_ref):
        rows = []
        for t in range(_T):
            h = jnp.zeros((BN, _HG), jnp.float32)
            for s in range(max(0, t - _WIN + 1), t + 1):
                xt = g_ref[s]
                gi = lax.dot_general(xt, wih_ref[...], (((1,), (0,)), ((), ())),
                                     preferred_element_type=jnp.float32) + bih_ref[...]
                gh = lax.dot_general(h, whh_ref[...], (((1,), (0,)), ((), ())),
                                     preferred_element_type=jnp.float32) + bhh_ref[...]
                r = jax.nn.sigmoid(gi[:, :_HG] + gh[:, :_HG])
                z = jax.nn.sigmoid(gi[:, _HG:2 * _HG] + gh[:, _HG:2 * _HG])
                n = jnp.tanh(gi[:, 2 * _HG:] + r * gh[:, 2 * _HG:])
                h = (1.0 - z) * n + z * h
            rows.append(jnp.sum(h * wp_ref[...], axis=1) + bp_ref[0, 0])
        o_ref[...] = jnp.stack(rows, axis=0)

    return pl.pallas_call(
        body,
        grid=(_N // BN,),
        in_specs=[
            pl.BlockSpec((_T, BN, _H1), lambda i: (0, i, 0)),
            pl.BlockSpec((_H1, 3 * _HG), lambda i: (0, 0)),
            pl.BlockSpec((_HG, 3 * _HG), lambda i: (0, 0)),
            pl.BlockSpec((1, 3 * _HG), lambda i: (0, 0)),
            pl.BlockSpec((1, 3 * _HG), lambda i: (0, 0)),
            pl.BlockSpec((1, _HG), lambda i: (0, 0)),
            pl.BlockSpec((1, 1), lambda i: (0, 0)),
        ],
        out_specs=pl.BlockSpec((_T, BN), lambda i: (0, i)),
        out_shape=jax.ShapeDtypeStruct((_T, _N), jnp.float32),
    )(gcn, wihT, whhT, bih, bhh, wpT, bp)


def kernel(x, edge_index, edge_weight, W1, b1, W2, b2,
           W_ih, W_hh, b_ih, b_hh, Wp, bp):
    src = edge_index[:, 0, :]
    dst = edge_index[:, 1, :]
    ew = edge_weight

    toff = (jnp.arange(_T, dtype=jnp.int32) * _N)[:, None]
    dstf = (dst + toff).reshape(_NW, _DCH, _CH)
    wf = ew.reshape(_NW, _DCH, _CH)
    z1 = jnp.zeros((_T * _N,), jnp.float32)
    degp = _sc_deg(dstf, wf, z1).reshape(_NC, _T, _N, 1)

    xs1_all, dis_all = _tc_pre(x, degp, W1)

    pad = _EP - _E
    srcp = jnp.pad(src, ((0, 0), (0, pad))).reshape(_T, _NW, _CHT, _CH)
    dstp = jnp.pad(dst, ((0, 0), (0, pad))).reshape(_T, _NW, _CHT, _CH)
    ewp = jnp.pad(ew, ((0, 0), (0, pad))).reshape(_T, _NW, _CHT, _CH)
    zrows = jnp.zeros((_N, _H1), jnp.float32)
    b1r = b1.reshape(1, _H1)
    b2r = b2.reshape(1, _H1)

    gcn_list = []
    for t in range(_T):
        parts1 = _sc_agg(xs1_all[t], srcp[t], dstp[t], ewp[t], zrows)
        xs2 = _tc_mid(parts1, xs1_all[t], dis_all[t], b1r, W2)
        parts2 = _sc_agg(xs2, srcp[t], dstp[t], ewp[t], zrows)
        gcn_list.append(_tc_post(parts2, xs2, dis_all[t], b2r))
    gcn = jnp.stack(gcn_list, axis=0)

    return _tc_gru(gcn, W_ih.T, W_hh.T, b_ih.reshape(1, 3 * _HG),
                   b_hh.reshape(1, 3 * _HG), Wp.T, bp.reshape(1, 1))


# speedup vs baseline: 7.3962x; 7.3962x over previous
import functools

import jax
import jax.numpy as jnp
from jax import lax
from jax.experimental import pallas as pl
from jax.experimental.pallas import tpu as pltpu
from jax.experimental.pallas import tpu_sc as plsc

_N = 10000
_T = 8
_E = 320000
_D = 128
_H1 = 64
_HG = 32
_WIN = 4

_NC = 2
_NS = 16
_NW = _NC * _NS
_CH = 128
_CHT = 80
_EP = _NW * _CHT * _CH
_NP = 10240
_DSLAB = 128
_DNS = 5
_DEP = _NW * _DNS * _DSLAB * _CH


def _sc_deg(dstf, wf, z1):
    mesh = plsc.VectorSubcoreMesh(core_axis_name="c", subcore_axis_name="s")

    @functools.partial(
        pl.kernel,
        mesh=mesh,
        out_type=jax.ShapeDtypeStruct((_NC * _T * _N,), jnp.float32),
        scratch_types=[
            pltpu.VMEM((_DSLAB, _CH), jnp.int32),
            pltpu.VMEM((_DSLAB, _CH), jnp.float32),
            pltpu.VMEM(((_T * _N) // _NS,), jnp.float32),
            pltpu.VMEM_SHARED((_T * _N,), jnp.float32),
        ],
    )
    def k(dst_hbm, w_hbm, z_hbm, out_hbm, dbuf, vbuf, bounce, acc):
        c = lax.axis_index("c")
        s = lax.axis_index("s")
        wid = s * _NC + c
        spt = (_T * _N) // _NS
        pltpu.sync_copy(z_hbm.at[pl.ds(s * spt, spt)], bounce)
        pltpu.sync_copy(bounce, acc.at[pl.ds(s * spt, spt)])
        plsc.subcore_barrier()
        for slab in range(_DNS):
            pltpu.sync_copy(dst_hbm.at[wid, slab], dbuf)
            pltpu.sync_copy(w_hbm.at[wid, slab], vbuf)

            def body(j, carry):
                pltpu.sync_copy(vbuf.at[j], acc.at[dbuf.at[j]], add=True)
                return carry

            lax.fori_loop(0, _DSLAB, body, 0)
        plsc.subcore_barrier()
        pltpu.sync_copy(acc.at[pl.ds(s * spt, spt)], bounce)
        pltpu.sync_copy(bounce, out_hbm.at[pl.ds(c * (_T * _N) + s * spt, spt)])

    return k(dstf, wf, z1)


def _sc_agg(xs, src, dst, w, zrows):
    mesh = plsc.VectorSubcoreMesh(core_axis_name="c", subcore_axis_name="s")

    @functools.partial(
        pl.kernel,
        mesh=mesh,
        out_type=jax.ShapeDtypeStruct((_NC, _NP, _H1), jnp.float32),
        scratch_types=[
            pltpu.VMEM((_CHT, _CH), jnp.int32),
            pltpu.VMEM((_CHT, _CH), jnp.int32),
            pltpu.VMEM((_CHT, _CH), jnp.float32),
            pltpu.VMEM((_CH, _H1), jnp.float32),
            pltpu.VMEM((_NP // _NS, _H1), jnp.float32),
            pltpu.VMEM_SHARED((_NP, _H1), jnp.float32),
            pltpu.SemaphoreType.DMA,
        ],
        compiler_params=pltpu.CompilerParams(use_tc_tiling_on_sc=False),
    )
    def k(xs_hbm, src_hbm, dst_hbm, w_hbm, z_hbm, out_hbm,
          srcb, dstb, wb, rows, bounce, acc, sem):
        c = lax.axis_index("c")
        s = lax.axis_index("s")
        wid = s * _NC + c
        rpt = _NP // _NS
        pltpu.sync_copy(src_hbm.at[wid], srcb)
        pltpu.sync_copy(dst_hbm.at[wid], dstb)
        pltpu.sync_copy(w_hbm.at[wid], wb)
        pltpu.sync_copy(z_hbm.at[pl.ds(s * rpt, rpt)], bounce)
        pltpu.sync_copy(bounce, acc.at[pl.ds(s * rpt, rpt)])
        plsc.subcore_barrier()

        def body(j, carry):
            pltpu.async_copy(xs_hbm.at[srcb.at[j]], rows, sem).wait()

            def scale(g, c2):
                wv = wb[j, pl.ds(g * 16, 16)]
                eb = g * 16
                for i in range(16):
                    ws = wv[i]
                    for f in range(_H1 // 16):
                        sl = pl.ds(f * 16, 16)
                        rows[eb + i, sl] = rows[eb + i, sl] * ws
                return c2

            lax.fori_loop(0, _CH // 16, scale, 0)
            pltpu.sync_copy(rows, acc.at[dstb.at[j]], add=True)
            return carry

        lax.fori_loop(0, _CHT, body, 0)
        plsc.subcore_barrier()
        pltpu.sync_copy(acc.at[pl.ds(s * rpt, rpt)], bounce)
        pltpu.sync_copy(bounce, out_hbm.at[c, pl.ds(s * rpt, rpt)])

    return k(xs, src, dst, w, zrows)


def _tc_pre(x, degp, W1):
    def body(x_ref, degp_ref, w1_ref, xs_ref, dis_ref):
        xt = x_ref[0]
        xw = lax.dot_general(xt, w1_ref[...], (((0,), (0,)), ((), ())),
                             preferred_element_type=jnp.float32)
        deg = degp_ref[0, 0] + degp_ref[1, 0] + 1.0
        dis = lax.rsqrt(deg)
        dis_ref[0] = dis
        xs_ref[0] = xw * dis

    return pl.pallas_call(
        body,
        grid=(_T,),
        in_specs=[
            pl.BlockSpec((1, _D, _N), lambda t: (t, 0, 0)),
            pl.BlockSpec((_NC, 1, _N, 1), lambda t: (0, t, 0, 0)),
            pl.BlockSpec((_D, _H1), lambda t: (0, 0)),
        ],
        out_specs=[
            pl.BlockSpec((1, _N, _H1), lambda t: (t, 0, 0)),
            pl.BlockSpec((1, _N, 1), lambda t: (t, 0, 0)),
        ],
        out_shape=[
            jax.ShapeDtypeStruct((_T, _N, _H1), jnp.float32),
            jax.ShapeDtypeStruct((_T, _N, 1), jnp.float32),
        ],
    )(x, degp, W1)


def _tc_mid(parts, xs1, dis, b1, W2):
    def body(p_ref, xs_ref, dis_ref, b_ref, w2_ref, o_ref):
        psum = p_ref[0] + p_ref[1] + xs_ref[...]
        out1 = jnp.maximum(psum * dis_ref[...] + b_ref[...], 0.0)
        xw2 = lax.dot_general(out1, w2_ref[...], (((1,), (0,)), ((), ())),
                              preferred_element_type=jnp.float32)
        o_ref[...] = xw2 * dis_ref[...]

    return pl.pallas_call(
        body,
        out_shape=jax.ShapeDtypeStruct((_N, _H1), jnp.float32),
    )(parts, xs1, dis, b1, W2)


def _tc_post(parts, xs2, dis, b2):
    def body(p_ref, xs_ref, dis_ref, b_ref, o_ref):
        psum = p_ref[0] + p_ref[1] + xs_ref[...]
        o_ref[...] = jnp.maximum(psum * dis_ref[...] + b_ref[...], 0.0)

    return pl.pallas_call(
        body,
        out_shape=jax.ShapeDtypeStruct((_N, _H1), jnp.float32),
    )(parts, xs2, dis, b2)


def _tc_gru(gcn, wihT, whhT, bih, bhh, wpT, bp):
    BN = 2000

    def body(g_ref, wih_ref, whh_ref, bih_ref, bhh_ref, wp_ref, bp_ref, o_ref):
        rows = []
        for t in range(_T):
            h = jnp.zeros((BN, _HG), jnp.float32)
            for s in range(max(0, t - _WIN + 1), t + 1):
                xt = g_ref[s]
                gi = lax.dot_general(xt, wih_ref[...], (((1,), (0,)), ((), ())),
                                     preferred_element_type=jnp.float32) + bih_ref[...]
                gh = lax.dot_general(h, whh_ref[...], (((1,), (0,)), ((), ())),
                                     preferred_element_type=jnp.float32) + bhh_ref[...]
                r = jax.nn.sigmoid(gi[:, :_HG] + gh[:, :_HG])
                z = jax.nn.sigmoid(gi[:, _HG:2 * _HG] + gh[:, _HG:2 * _HG])
                n = jnp.tanh(gi[:, 2 * _HG:] + r * gh[:, 2 * _HG:])
                h = (1.0 - z) * n + z * h
            rows.append(jnp.sum(h * wp_ref[...], axis=1, keepdims=True)
                        + bp_ref[0, 0])
        o_ref[...] = jnp.concatenate(rows, axis=1)

    return pl.pallas_call(
        body,
        grid=(_N // BN,),
        in_specs=[
            pl.BlockSpec((_T, BN, _H1), lambda i: (0, i, 0)),
            pl.BlockSpec((_H1, 3 * _HG), lambda i: (0, 0)),
            pl.BlockSpec((_HG, 3 * _HG), lambda i: (0, 0)),
            pl.BlockSpec((1, 3 * _HG), lambda i: (0, 0)),
            pl.BlockSpec((1, 3 * _HG), lambda i: (0, 0)),
            pl.BlockSpec((1, _HG), lambda i: (0, 0)),
            pl.BlockSpec((1, 1), lambda i: (0, 0)),
        ],
        out_specs=pl.BlockSpec((BN, _T), lambda i: (i, 0)),
        out_shape=jax.ShapeDtypeStruct((_N, _T), jnp.float32),
    )(gcn, wihT, whhT, bih, bhh, wpT, bp)


def kernel(x, edge_index, edge_weight, W1, b1, W2, b2,
           W_ih, W_hh, b_ih, b_hh, Wp, bp):
    src = edge_index[:, 0, :]
    dst = edge_index[:, 1, :]
    ew = edge_weight

    toff = (jnp.arange(_T, dtype=jnp.int32) * _N)[:, None]
    dpad = _DEP - _T * _E
    dstf = jnp.pad((dst + toff).reshape(-1), (0, dpad)).reshape(
        _NW, _DNS, _DSLAB, _CH)
    wf = jnp.pad(ew.reshape(-1), (0, dpad)).reshape(_NW, _DNS, _DSLAB, _CH)
    z1 = jnp.zeros((_T * _N,), jnp.float32)
    degp = _sc_deg(dstf, wf, z1).reshape(_NC, _T, _N, 1)

    xs1_all, dis_all = _tc_pre(x, degp, W1)

    pad = _EP - _E
    srcp = jnp.pad(src, ((0, 0), (0, pad))).reshape(_T, _NW, _CHT, _CH)
    dstp = jnp.pad(dst, ((0, 0), (0, pad))).reshape(_T, _NW, _CHT, _CH)
    ewp = jnp.pad(ew, ((0, 0), (0, pad))).reshape(_T, _NW, _CHT, _CH)
    zrows = jnp.zeros((_NP, _H1), jnp.float32)
    b1r = b1.reshape(1, _H1)
    b2r = b2.reshape(1, _H1)

    gcn_list = []
    for t in range(_T):
        parts1 = _sc_agg(xs1_all[t], srcp[t], dstp[t], ewp[t], zrows)
        xs2 = _tc_mid(parts1[:, :_N], xs1_all[t], dis_all[t], b1r, W2)
        parts2 = _sc_agg(xs2, srcp[t], dstp[t], ewp[t], zrows)
        gcn_list.append(_tc_post(parts2[:, :_N], xs2, dis_all[t], b2r))
    gcn = jnp.stack(gcn_list, axis=0)

    preds_nt = _tc_gru(gcn, W_ih.T, W_hh.T, b_ih.reshape(1, 3 * _HG),
                       b_hh.reshape(1, 3 * _HG), Wp.T, bp.reshape(1, 1))
    return preds_nt.T


# speedup vs baseline: 9.8663x; 1.3340x over previous
import functools

import jax
import jax.numpy as jnp
from jax import lax
from jax.experimental import pallas as pl
from jax.experimental.pallas import tpu as pltpu
from jax.experimental.pallas import tpu_sc as plsc

_N = 10000
_T = 8
_E = 320000
_D = 128
_H1 = 64
_HG = 32
_WIN = 4

_NC = 2
_NS = 16
_NW = _NC * _NS
_NB = 10240
_FPT = _H1 // _NS
_CE = 2048
_ECH = 80
_DCH = -(-(_T * _E) // (_NW * _CE))


def _sc_deg(dstf, wf, z1):
    mesh = plsc.VectorSubcoreMesh(core_axis_name="c", subcore_axis_name="s")

    @functools.partial(
        pl.kernel,
        mesh=mesh,
        out_type=jax.ShapeDtypeStruct((_NW * _T * _NB,), jnp.float32),
        scratch_types=[
            pltpu.VMEM((_T * _NB,), jnp.float32),
            pltpu.VMEM((_CE,), jnp.int32),
            pltpu.VMEM((_CE,), jnp.float32),
            pltpu.VMEM((_CE,), jnp.int32),
            pltpu.VMEM((_CE,), jnp.float32),
            pltpu.SemaphoreType.DMA,
            pltpu.SemaphoreType.DMA,
        ],
        compiler_params=pltpu.CompilerParams(use_tc_tiling_on_sc=False, needs_layout_passes=False),
    )
    def k(dst_hbm, w_hbm, z_hbm, out_hbm,
          acc, dbA, wbA, dbB, wbB, semA, semB):
        c = lax.axis_index("c")
        s = lax.axis_index("s")
        wid = s * _NC + c
        pltpu.sync_copy(z_hbm, acc)

        def load(j, db, wb, sem):
            pltpu.async_copy(dst_hbm.at[wid, j], db, sem)
            pltpu.async_copy(w_hbm.at[wid, j], wb, sem)

        def drain(db, wb, sem):
            pltpu.make_async_copy(dst_hbm.at[wid, 0], db, sem).wait()
            pltpu.make_async_copy(w_hbm.at[wid, 0], wb, sem).wait()

        def process(db, wb):
            def grp(g, carry):
                i16 = db[pl.ds(g * 16, 16)]
                w16 = wb[pl.ds(g * 16, 16)]
                plsc.addupdate_scatter(acc, [i16], w16)
                return carry

            lax.fori_loop(0, _CE // 16, grp, 0)

        load(0, dbA, wbA, semA)

        def body(j2, carry):
            jA = 2 * j2
            load(jA + 1, dbB, wbB, semB)
            drain(dbA, wbA, semA)
            process(dbA, wbA)

            @pl.when(j2 < _DCH // 2 - 1)
            def _():
                load(jA + 2, dbA, wbA, semA)

            drain(dbB, wbB, semB)
            process(dbB, wbB)
            return carry

        lax.fori_loop(0, _DCH // 2, body, 0)
        pltpu.sync_copy(acc, out_hbm.at[pl.ds(wid * (_T * _NB), _T * _NB)])

    return k(dstf, wf, z1)


def _sc_agg(xsT, src, dst, w, zrow):
    mesh = plsc.VectorSubcoreMesh(core_axis_name="c", subcore_axis_name="s")

    @functools.partial(
        pl.kernel,
        mesh=mesh,
        out_type=jax.ShapeDtypeStruct((_NC * _T * _H1 * _NB,), jnp.float32),
        scratch_types=[
            [pltpu.VMEM((_NB,), jnp.float32) for _ in range(_FPT)],
            [pltpu.VMEM((_NB,), jnp.float32) for _ in range(_FPT)],
            pltpu.VMEM((_CE,), jnp.int32),
            pltpu.VMEM((_CE,), jnp.int32),
            pltpu.VMEM((_CE,), jnp.float32),
            pltpu.VMEM((_CE,), jnp.int32),
            pltpu.VMEM((_CE,), jnp.int32),
            pltpu.VMEM((_CE,), jnp.float32),
            pltpu.SemaphoreType.DMA,
            pltpu.SemaphoreType.DMA,
        ],
        compiler_params=pltpu.CompilerParams(use_tc_tiling_on_sc=False, needs_layout_passes=False),
    )
    def k(xs_hbm, src_hbm, dst_hbm, w_hbm, z_hbm, out_hbm,
          xsv, accv, sbA, dbA, wbA, sbB, dbB, wbB, semA, semB):
        c = lax.axis_index("c")
        s = lax.axis_index("s")

        for t in range(_T):
            for i in range(_FPT):
                fi = _FPT * s + i
                pltpu.sync_copy(xs_hbm.at[t, fi], xsv[i])
                pltpu.sync_copy(z_hbm, accv[i])

            def load(j, sb, db, wb, sem):
                pltpu.async_copy(src_hbm.at[t, c, j], sb, sem)
                pltpu.async_copy(dst_hbm.at[t, c, j], db, sem)
                pltpu.async_copy(w_hbm.at[t, c, j], wb, sem)

            def drain(sb, db, wb, sem):
                pltpu.make_async_copy(src_hbm.at[t, c, 0], sb, sem).wait()
                pltpu.make_async_copy(dst_hbm.at[t, c, 0], db, sem).wait()
                pltpu.make_async_copy(w_hbm.at[t, c, 0], wb, sem).wait()

            def process(sb, db, wb):
                def grp(g, carry):
                    s16 = sb[pl.ds(g * 16, 16)]
                    d16 = db[pl.ds(g * 16, 16)]
                    w16 = wb[pl.ds(g * 16, 16)]
                    for i in range(_FPT):
                        v = plsc.load_gather(xsv[i], [s16])
                        plsc.addupdate_scatter(accv[i], [d16], v * w16)
                    return carry

                lax.fori_loop(0, _CE // 16, grp, 0)

            load(0, sbA, dbA, wbA, semA)

            def body(j2, carry):
                jA = 2 * j2
                load(jA + 1, sbB, dbB, wbB, semB)
                drain(sbA, dbA, wbA, semA)
                process(sbA, dbA, wbA)

                @pl.when(j2 < _ECH // 2 - 1)
                def _():
                    load(jA + 2, sbA, dbA, wbA, semA)

                drain(sbB, dbB, wbB, semB)
                process(sbB, dbB, wbB)
                return carry

            lax.fori_loop(0, _ECH // 2, body, 0)

            for i in range(_FPT):
                fi = _FPT * s + i
                off = ((c * _T + t) * _H1 + fi) * _NB
                pltpu.sync_copy(accv[i], out_hbm.at[pl.ds(off, _NB)])

    return k(xsT, src, dst, w, zrow)


def _tc_pre(x, degp, W1):
    def body(x_ref, degp_ref, w1_ref, xs_ref, dis_ref):
        xt = x_ref[0]
        xw = lax.dot_general(w1_ref[...], xt, (((0,), (0,)), ((), ())),
                             preferred_element_type=jnp.float32)
        deg = jnp.sum(degp_ref[:, 0, 0, :], axis=0, keepdims=True) + 1.0
        dis = lax.rsqrt(deg)
        dis_ref[0] = dis
        xs_ref[0] = xw * dis

    return pl.pallas_call(
        body,
        grid=(_T,),
        in_specs=[
            pl.BlockSpec((1, _D, _NB), lambda t: (t, 0, 0)),
            pl.BlockSpec((_NW, 1, 1, _NB), lambda t: (0, t, 0, 0)),
            pl.BlockSpec((_D, _H1), lambda t: (0, 0)),
        ],
        out_specs=[
            pl.BlockSpec((1, _H1, _NB), lambda t: (t, 0, 0)),
            pl.BlockSpec((1, 1, _NB), lambda t: (t, 0, 0)),
        ],
        out_shape=[
            jax.ShapeDtypeStruct((_T, _H1, _NB), jnp.float32),
            jax.ShapeDtypeStruct((_T, 1, _NB), jnp.float32),
        ],
    )(x, degp, W1)


def _tc_mid(parts, xs1, dis, b1, W2):
    def body(p_ref, xs_ref, dis_ref, b_ref, w2_ref, o_ref):
        d = dis_ref[0]
        psum = p_ref[0, 0] + p_ref[1, 0] + xs_ref[0]
        out1 = jnp.maximum(psum * d + b_ref[...], 0.0)
        xw2 = lax.dot_general(w2_ref[...], out1, (((0,), (0,)), ((), ())),
                              preferred_element_type=jnp.float32)
        o_ref[0] = xw2 * d

    return pl.pallas_call(
        body,
        grid=(_T,),
        in_specs=[
            pl.BlockSpec((_NC, 1, _H1, _NB), lambda t: (0, t, 0, 0)),
            pl.BlockSpec((1, _H1, _NB), lambda t: (t, 0, 0)),
            pl.BlockSpec((1, 1, _NB), lambda t: (t, 0, 0)),
            pl.BlockSpec((_H1, 1), lambda t: (0, 0)),
            pl.BlockSpec((_H1, _H1), lambda t: (0, 0)),
        ],
        out_specs=pl.BlockSpec((1, _H1, _NB), lambda t: (t, 0, 0)),
        out_shape=jax.ShapeDtypeStruct((_T, _H1, _NB), jnp.float32),
    )(parts, xs1, dis, b1, W2)


def _tc_post(parts, xs2, dis, b2):
    def body(p_ref, xs_ref, dis_ref, b_ref, o_ref):
        psum = p_ref[0, 0] + p_ref[1, 0] + xs_ref[0]
        o_ref[0] = jnp.maximum(psum * dis_ref[0] + b_ref[...], 0.0)

    return pl.pallas_call(
        body,
        grid=(_T,),
        in_specs=[
            pl.BlockSpec((_NC, 1, _H1, _NB), lambda t: (0, t, 0, 0)),
            pl.BlockSpec((1, _H1, _NB), lambda t: (t, 0, 0)),
            pl.BlockSpec((1, 1, _NB), lambda t: (t, 0, 0)),
            pl.BlockSpec((_H1, 1), lambda t: (0, 0)),
        ],
        out_specs=pl.BlockSpec((1, _H1, _NB), lambda t: (t, 0, 0)),
        out_shape=jax.ShapeDtypeStruct((_T, _H1, _NB), jnp.float32),
    )(parts, xs2, dis, b2)


def _tc_gru(gcn, wihT, whhT, bih, bhh, wpT, bp):
    BN = 2048

    def body(g_ref, wih_ref, whh_ref, bih_ref, bhh_ref, wp_ref, bp_ref, o_ref):
        rows = []
        for t in range(_T):
            h = jnp.zeros((BN, _HG), jnp.float32)
            for s in range(max(0, t - _WIN + 1), t + 1):
                gi = lax.dot_general(g_ref[s], wih_ref[...],
                                     (((0,), (0,)), ((), ())),
                                     preferred_element_type=jnp.float32) + bih_ref[...]
                gh = lax.dot_general(h, whh_ref[...], (((1,), (0,)), ((), ())),
                                     preferred_element_type=jnp.float32) + bhh_ref[...]
                r = jax.nn.sigmoid(gi[:, :_HG] + gh[:, :_HG])
                z = jax.nn.sigmoid(gi[:, _HG:2 * _HG] + gh[:, _HG:2 * _HG])
                n = jnp.tanh(gi[:, 2 * _HG:] + r * gh[:, 2 * _HG:])
                h = (1.0 - z) * n + z * h
            rows.append(jnp.sum(h * wp_ref[...], axis=1, keepdims=True)
                        + bp_ref[0, 0])
        o_ref[...] = jnp.concatenate(rows, axis=1)

    return pl.pallas_call(
        body,
        grid=(_NB // BN,),
        in_specs=[
            pl.BlockSpec((_T, _H1, BN), lambda i: (0, 0, i)),
            pl.BlockSpec((_H1, 3 * _HG), lambda i: (0, 0)),
            pl.BlockSpec((_HG, 3 * _HG), lambda i: (0, 0)),
            pl.BlockSpec((1, 3 * _HG), lambda i: (0, 0)),
            pl.BlockSpec((1, 3 * _HG), lambda i: (0, 0)),
            pl.BlockSpec((1, _HG), lambda i: (0, 0)),
            pl.BlockSpec((1, 1), lambda i: (0, 0)),
        ],
        out_specs=pl.BlockSpec((BN, _T), lambda i: (i, 0)),
        out_shape=jax.ShapeDtypeStruct((_NB, _T), jnp.float32),
    )(gcn, wihT, whhT, bih, bhh, wpT, bp)


def kernel(x, edge_index, edge_weight, W1, b1, W2, b2,
           W_ih, W_hh, b_ih, b_hh, Wp, bp):
    src = edge_index[:, 0, :]
    dst = edge_index[:, 1, :]
    ew = edge_weight
    xpad = jnp.pad(x, ((0, 0), (0, 0), (0, _NB - _N)))

    toff = (jnp.arange(_T, dtype=jnp.int32) * _NB)[:, None]
    dpad = _NW * _DCH * _CE - _T * _E
    dstf = jnp.pad((dst + toff).reshape(-1), (0, dpad)).reshape(_NW, _DCH, _CE)
    wf = jnp.pad(ew.reshape(-1), (0, dpad)).reshape(_NW, _DCH, _CE)
    z1 = jnp.zeros((_T * _NB,), jnp.float32)
    degp = _sc_deg(dstf, wf, z1).reshape(_NW, _T, 1, _NB)

    xs1_all, dis_all = _tc_pre(xpad, degp, W1)

    half = _E // _NC
    epad = _ECH * _CE - half
    srcp = jnp.pad(src.reshape(_T, _NC, half), ((0, 0), (0, 0), (0, epad))
                   ).reshape(_T, _NC, _ECH, _CE)
    dstp = jnp.pad(dst.reshape(_T, _NC, half), ((0, 0), (0, 0), (0, epad))
                   ).reshape(_T, _NC, _ECH, _CE)
    ewp = jnp.pad(ew.reshape(_T, _NC, half), ((0, 0), (0, 0), (0, epad))
                  ).reshape(_T, _NC, _ECH, _CE)
    zrow = jnp.zeros((_NB,), jnp.float32)
    b1c = b1.reshape(_H1, 1)
    b2c = b2.reshape(_H1, 1)

    parts1 = _sc_agg(xs1_all, srcp, dstp, ewp, zrow).reshape(
        _NC, _T, _H1, _NB)
    xs2_all = _tc_mid(parts1, xs1_all, dis_all, b1c, W2)

    parts2 = _sc_agg(xs2_all, srcp, dstp, ewp, zrow).reshape(
        _NC, _T, _H1, _NB)
    gcn = _tc_post(parts2, xs2_all, dis_all, b2c)

    preds_nt = _tc_gru(gcn, W_ih.T, W_hh.T, b_ih.reshape(1, 3 * _HG),
                       b_hh.reshape(1, 3 * _HG), Wp.T, bp.reshape(1, 1))
    return preds_nt.T[:, :_N]


# speedup vs baseline: 20.6588x; 2.0939x over previous
import functools

import jax
import jax.numpy as jnp
from jax import lax
from jax.experimental import pallas as pl
from jax.experimental.pallas import tpu as pltpu
from jax.experimental.pallas import tpu_sc as plsc

_N = 10000
_T = 8
_E = 320000
_D = 128
_H1 = 64
_HG = 32
_WIN = 4

_NC = 2
_NS = 16
_NW = _NC * _NS
_NB = 10240
_FPT = _H1 // _NS
_CE = 2048
_ECH = 80
_DCH = -(-(_T * _E) // (_NW * _CE))


def _sc_deg(dstf, wf, z1):
    mesh = plsc.VectorSubcoreMesh(core_axis_name="c", subcore_axis_name="s")

    @functools.partial(
        pl.kernel,
        mesh=mesh,
        out_type=jax.ShapeDtypeStruct((_NW * _T * _NB,), jnp.float32),
        scratch_types=[
            pltpu.VMEM((_T * _NB,), jnp.float32),
            pltpu.VMEM((_CE,), jnp.int32),
            pltpu.VMEM((_CE,), jnp.float32),
            pltpu.VMEM((_CE,), jnp.int32),
            pltpu.VMEM((_CE,), jnp.float32),
            pltpu.SemaphoreType.DMA,
            pltpu.SemaphoreType.DMA,
        ],
        compiler_params=pltpu.CompilerParams(use_tc_tiling_on_sc=False, needs_layout_passes=False),
    )
    def k(dst_hbm, w_hbm, z_hbm, out_hbm,
          acc, dbA, wbA, dbB, wbB, semA, semB):
        c = lax.axis_index("c")
        s = lax.axis_index("s")
        wid = s * _NC + c
        pltpu.sync_copy(z_hbm, acc)

        def load(j, db, wb, sem):
            pltpu.async_copy(dst_hbm.at[wid, j], db, sem)
            pltpu.async_copy(w_hbm.at[wid, j], wb, sem)

        def drain(db, wb, sem):
            pltpu.make_async_copy(dst_hbm.at[wid, 0], db, sem).wait()
            pltpu.make_async_copy(w_hbm.at[wid, 0], wb, sem).wait()

        def process(db, wb):
            @plsc.parallel_loop(0, _CE // 16, unroll=8)
            def grp(g):
                i16 = db[pl.ds(g * 16, 16)]
                w16 = wb[pl.ds(g * 16, 16)]
                plsc.addupdate_scatter(acc, [i16], w16)

        load(0, dbA, wbA, semA)

        def body(j2, carry):
            jA = 2 * j2
            load(jA + 1, dbB, wbB, semB)
            drain(dbA, wbA, semA)
            process(dbA, wbA)

            @pl.when(j2 < _DCH // 2 - 1)
            def _():
                load(jA + 2, dbA, wbA, semA)

            drain(dbB, wbB, semB)
            process(dbB, wbB)
            return carry

        lax.fori_loop(0, _DCH // 2, body, 0)
        pltpu.sync_copy(acc, out_hbm.at[pl.ds(wid * (_T * _NB), _T * _NB)])

    return k(dstf, wf, z1)


def _sc_agg(xsT, src, dst, w, zrow):
    mesh = plsc.VectorSubcoreMesh(core_axis_name="c", subcore_axis_name="s")

    @functools.partial(
        pl.kernel,
        mesh=mesh,
        out_type=jax.ShapeDtypeStruct((_NC * _T * _H1 * _NB,), jnp.float32),
        scratch_types=[
            [pltpu.VMEM((_NB,), jnp.float32) for _ in range(_FPT)],
            [pltpu.VMEM((_NB,), jnp.float32) for _ in range(_FPT)],
            pltpu.VMEM((_CE,), jnp.int32),
            pltpu.VMEM((_CE,), jnp.int32),
            pltpu.VMEM((_CE,), jnp.float32),
            pltpu.VMEM((_CE,), jnp.int32),
            pltpu.VMEM((_CE,), jnp.int32),
            pltpu.VMEM((_CE,), jnp.float32),
            pltpu.SemaphoreType.DMA,
            pltpu.SemaphoreType.DMA,
        ],
        compiler_params=pltpu.CompilerParams(use_tc_tiling_on_sc=False, needs_layout_passes=False),
    )
    def k(xs_hbm, src_hbm, dst_hbm, w_hbm, z_hbm, out_hbm,
          xsv, accv, sbA, dbA, wbA, sbB, dbB, wbB, semA, semB):
        c = lax.axis_index("c")
        s = lax.axis_index("s")

        for t in range(_T):
            for i in range(_FPT):
                fi = _FPT * s + i
                pltpu.sync_copy(xs_hbm.at[t, fi], xsv[i])
                pltpu.sync_copy(z_hbm, accv[i])

            def load(j, sb, db, wb, sem):
                pltpu.async_copy(src_hbm.at[t, c, j], sb, sem)
                pltpu.async_copy(dst_hbm.at[t, c, j], db, sem)
                pltpu.async_copy(w_hbm.at[t, c, j], wb, sem)

            def drain(sb, db, wb, sem):
                pltpu.make_async_copy(src_hbm.at[t, c, 0], sb, sem).wait()
                pltpu.make_async_copy(dst_hbm.at[t, c, 0], db, sem).wait()
                pltpu.make_async_copy(w_hbm.at[t, c, 0], wb, sem).wait()

            def process(sb, db, wb):
                @plsc.parallel_loop(0, _CE // 16, unroll=8)
                def grp(g):
                    s16 = sb[pl.ds(g * 16, 16)]
                    d16 = db[pl.ds(g * 16, 16)]
                    w16 = wb[pl.ds(g * 16, 16)]
                    for i in range(_FPT):
                        v = plsc.load_gather(xsv[i], [s16])
                        plsc.addupdate_scatter(accv[i], [d16], v * w16)

            load(0, sbA, dbA, wbA, semA)

            def body(j2, carry):
                jA = 2 * j2
                load(jA + 1, sbB, dbB, wbB, semB)
                drain(sbA, dbA, wbA, semA)
                process(sbA, dbA, wbA)

                @pl.when(j2 < _ECH // 2 - 1)
                def _():
                    load(jA + 2, sbA, dbA, wbA, semA)

                drain(sbB, dbB, wbB, semB)
                process(sbB, dbB, wbB)
                return carry

            lax.fori_loop(0, _ECH // 2, body, 0)

            for i in range(_FPT):
                fi = _FPT * s + i
                off = ((c * _T + t) * _H1 + fi) * _NB
                pltpu.sync_copy(accv[i], out_hbm.at[pl.ds(off, _NB)])

    return k(xsT, src, dst, w, zrow)


def _tc_pre(x, degp, W1):
    def body(x_ref, degp_ref, w1_ref, xs_ref, dis_ref):
        xt = x_ref[0]
        xw = lax.dot_general(w1_ref[...], xt, (((0,), (0,)), ((), ())),
                             preferred_element_type=jnp.float32)
        deg = jnp.sum(degp_ref[:, 0, 0, :], axis=0, keepdims=True) + 1.0
        dis = lax.rsqrt(deg)
        dis_ref[0] = dis
        xs_ref[0] = xw * dis

    return pl.pallas_call(
        body,
        grid=(_T,),
        in_specs=[
            pl.BlockSpec((1, _D, _NB), lambda t: (t, 0, 0)),
            pl.BlockSpec((_NW, 1, 1, _NB), lambda t: (0, t, 0, 0)),
            pl.BlockSpec((_D, _H1), lambda t: (0, 0)),
        ],
        out_specs=[
            pl.BlockSpec((1, _H1, _NB), lambda t: (t, 0, 0)),
            pl.BlockSpec((1, 1, _NB), lambda t: (t, 0, 0)),
        ],
        out_shape=[
            jax.ShapeDtypeStruct((_T, _H1, _NB), jnp.float32),
            jax.ShapeDtypeStruct((_T, 1, _NB), jnp.float32),
        ],
    )(x, degp, W1)


def _tc_mid(parts, xs1, dis, b1, W2):
    def body(p_ref, xs_ref, dis_ref, b_ref, w2_ref, o_ref):
        d = dis_ref[0]
        psum = p_ref[0, 0] + p_ref[1, 0] + xs_ref[0]
        out1 = jnp.maximum(psum * d + b_ref[...], 0.0)
        xw2 = lax.dot_general(w2_ref[...], out1, (((0,), (0,)), ((), ())),
                              preferred_element_type=jnp.float32)
        o_ref[0] = xw2 * d

    return pl.pallas_call(
        body,
        grid=(_T,),
        in_specs=[
            pl.BlockSpec((_NC, 1, _H1, _NB), lambda t: (0, t, 0, 0)),
            pl.BlockSpec((1, _H1, _NB), lambda t: (t, 0, 0)),
            pl.BlockSpec((1, 1, _NB), lambda t: (t, 0, 0)),
            pl.BlockSpec((_H1, 1), lambda t: (0, 0)),
            pl.BlockSpec((_H1, _H1), lambda t: (0, 0)),
        ],
        out_specs=pl.BlockSpec((1, _H1, _NB), lambda t: (t, 0, 0)),
        out_shape=jax.ShapeDtypeStruct((_T, _H1, _NB), jnp.float32),
    )(parts, xs1, dis, b1, W2)


def _tc_post(parts, xs2, dis, b2):
    def body(p_ref, xs_ref, dis_ref, b_ref, o_ref):
        psum = p_ref[0, 0] + p_ref[1, 0] + xs_ref[0]
        o_ref[0] = jnp.maximum(psum * dis_ref[0] + b_ref[...], 0.0)

    return pl.pallas_call(
        body,
        grid=(_T,),
        in_specs=[
            pl.BlockSpec((_NC, 1, _H1, _NB), lambda t: (0, t, 0, 0)),
            pl.BlockSpec((1, _H1, _NB), lambda t: (t, 0, 0)),
            pl.BlockSpec((1, 1, _NB), lambda t: (t, 0, 0)),
            pl.BlockSpec((_H1, 1), lambda t: (0, 0)),
        ],
        out_specs=pl.BlockSpec((1, _H1, _NB), lambda t: (t, 0, 0)),
        out_shape=jax.ShapeDtypeStruct((_T, _H1, _NB), jnp.float32),
    )(parts, xs2, dis, b2)


def _tc_gru(gcn, wihT, whhT, bih, bhh, wpT, bp):
    BN = 2048

    def body(g_ref, wih_ref, whh_ref, bih_ref, bhh_ref, wp_ref, bp_ref, o_ref):
        rows = []
        for t in range(_T):
            h = jnp.zeros((BN, _HG), jnp.float32)
            for s in range(max(0, t - _WIN + 1), t + 1):
                gi = lax.dot_general(g_ref[s], wih_ref[...],
                                     (((0,), (0,)), ((), ())),
                                     preferred_element_type=jnp.float32) + bih_ref[...]
                gh = lax.dot_general(h, whh_ref[...], (((1,), (0,)), ((), ())),
                                     preferred_element_type=jnp.float32) + bhh_ref[...]
                r = jax.nn.sigmoid(gi[:, :_HG] + gh[:, :_HG])
                z = jax.nn.sigmoid(gi[:, _HG:2 * _HG] + gh[:, _HG:2 * _HG])
                n = jnp.tanh(gi[:, 2 * _HG:] + r * gh[:, 2 * _HG:])
                h = (1.0 - z) * n + z * h
            rows.append(jnp.sum(h * wp_ref[...], axis=1, keepdims=True)
                        + bp_ref[0, 0])
        o_ref[...] = jnp.concatenate(rows, axis=1)

    return pl.pallas_call(
        body,
        grid=(_NB // BN,),
        in_specs=[
            pl.BlockSpec((_T, _H1, BN), lambda i: (0, 0, i)),
            pl.BlockSpec((_H1, 3 * _HG), lambda i: (0, 0)),
            pl.BlockSpec((_HG, 3 * _HG), lambda i: (0, 0)),
            pl.BlockSpec((1, 3 * _HG), lambda i: (0, 0)),
            pl.BlockSpec((1, 3 * _HG), lambda i: (0, 0)),
            pl.BlockSpec((1, _HG), lambda i: (0, 0)),
            pl.BlockSpec((1, 1), lambda i: (0, 0)),
        ],
        out_specs=pl.BlockSpec((BN, _T), lambda i: (i, 0)),
        out_shape=jax.ShapeDtypeStruct((_NB, _T), jnp.float32),
    )(gcn, wihT, whhT, bih, bhh, wpT, bp)


def kernel(x, edge_index, edge_weight, W1, b1, W2, b2,
           W_ih, W_hh, b_ih, b_hh, Wp, bp):
    src = edge_index[:, 0, :]
    dst = edge_index[:, 1, :]
    ew = edge_weight
    xpad = jnp.pad(x, ((0, 0), (0, 0), (0, _NB - _N)))

    toff = (jnp.arange(_T, dtype=jnp.int32) * _NB)[:, None]
    dpad = _NW * _DCH * _CE - _T * _E
    dstf = jnp.pad((dst + toff).reshape(-1), (0, dpad)).reshape(_NW, _DCH, _CE)
    wf = jnp.pad(ew.reshape(-1), (0, dpad)).reshape(_NW, _DCH, _CE)
    z1 = jnp.zeros((_T * _NB,), jnp.float32)
    degp = _sc_deg(dstf, wf, z1).reshape(_NW, _T, 1, _NB)

    xs1_all, dis_all = _tc_pre(xpad, degp, W1)

    half = _E // _NC
    epad = _ECH * _CE - half
    srcp = jnp.pad(src.reshape(_T, _NC, half), ((0, 0), (0, 0), (0, epad))
                   ).reshape(_T, _NC, _ECH, _CE)
    dstp = jnp.pad(dst.reshape(_T, _NC, half), ((0, 0), (0, 0), (0, epad))
                   ).reshape(_T, _NC, _ECH, _CE)
    ewp = jnp.pad(ew.reshape(_T, _NC, half), ((0, 0), (0, 0), (0, epad))
                  ).reshape(_T, _NC, _ECH, _CE)
    zrow = jnp.zeros((_NB,), jnp.float32)
    b1c = b1.reshape(_H1, 1)
    b2c = b2.reshape(_H1, 1)

    parts1 = _sc_agg(xs1_all, srcp, dstp, ewp, zrow).reshape(
        _NC, _T, _H1, _NB)
    xs2_all = _tc_mid(parts1, xs1_all, dis_all, b1c, W2)

    parts2 = _sc_agg(xs2_all, srcp, dstp, ewp, zrow).reshape(
        _NC, _T, _H1, _NB)
    gcn = _tc_post(parts2, xs2_all, dis_all, b2c)

    preds_nt = _tc_gru(gcn, W_ih.T, W_hh.T, b_ih.reshape(1, 3 * _HG),
                       b_hh.reshape(1, 3 * _HG), Wp.T, bp.reshape(1, 1))
    return preds_nt.T[:, :_N]
